# baseline (speedup 1.0000x reference)
import functools

import jax
import jax.numpy as jnp
from jax import lax
from jax.experimental import pallas as pl

B = 256
C = 768
HW = 196
NUM_CLASSES = 1000
E = 16
TOP_K = 2

BB = 16


def _router_body(hs_ref, task_ref, w1_ref, w2_ref, b_ref,
                 logits_ref, weights_ref, sel_ref, mask_ref):
    x = hs_ref[...]
    img = jnp.mean(x, axis=-1)

    t = task_ref[...]
    t = t - jnp.max(t, axis=-1, keepdims=True)
    te = jnp.exp(t)
    tsm = te / jnp.sum(te, axis=-1, keepdims=True)

    dn = (((1,), (1,)), ((), ()))
    logits = (lax.dot_general(img, w1_ref[...], dn,
                              preferred_element_type=jnp.float32)
              + lax.dot_general(tsm, w2_ref[...], dn,
                                preferred_element_type=jnp.float32)
              + b_ref[...])
    logits_ref[...] = logits

    m = jnp.max(logits, axis=-1, keepdims=True)
    pe = jnp.exp(logits - m)
    probs = pe / jnp.sum(pe, axis=-1, keepdims=True)

    lane = lax.broadcasted_iota(jnp.int32, (BB, E), 1)
    v1 = jnp.max(probs, axis=-1, keepdims=True)
    i1 = jnp.min(jnp.where(probs == v1, lane, E), axis=-1, keepdims=True)
    p2 = jnp.where(lane == i1, -jnp.inf, probs)
    v2 = jnp.max(p2, axis=-1, keepdims=True)
    i2 = jnp.min(jnp.where(p2 == v2, lane, E), axis=-1, keepdims=True)

    s = v1 + v2
    weights_ref[...] = jnp.concatenate([v1 / s, v2 / s], axis=1)
    sel_ref[...] = jnp.concatenate([i1, i2], axis=1)

    logits_t = (lax.dot_general(w1_ref[...], img, dn,
                                preferred_element_type=jnp.float32)
                + lax.dot_general(w2_ref[...], tsm, dn,
                                  preferred_element_type=jnp.float32)
                + b_ref[...].reshape(E, 1))
    mt = jnp.max(logits_t, axis=0, keepdims=True)
    pet = jnp.exp(logits_t - mt)
    probs_t = pet / jnp.sum(pet, axis=0, keepdims=True)
    erow = lax.broadcasted_iota(jnp.int32, (E, BB), 0)
    v1t = jnp.max(probs_t, axis=0, keepdims=True)
    i1t = jnp.min(jnp.where(probs_t == v1t, erow, E), axis=0, keepdims=True)
    p2t = jnp.where(erow == i1t, -jnp.inf, probs_t)
    v2t = jnp.max(p2t, axis=0, keepdims=True)
    i2t = jnp.min(jnp.where(p2t == v2t, erow, E), axis=0, keepdims=True)

    e3 = lax.broadcasted_iota(jnp.int32, (1, E, TOP_K, BB), 1)
    k3 = lax.broadcasted_iota(jnp.int32, (1, E, TOP_K, BB), 2)
    sel3 = jnp.where(k3 == 0, i1t.reshape(1, 1, 1, BB), i2t.reshape(1, 1, 1, BB))
    mask_ref[...] = (e3 == sel3).astype(jnp.int32)


@functools.partial(jax.jit, static_argnames=("interpret",))
def _run(hs3, task_cls, w1, w2, b2, interpret=False):
    grid = (B // BB,)
    return pl.pallas_call(
        _router_body,
        grid=grid,
        in_specs=[
            pl.BlockSpec((BB, C, HW), lambda i: (i, 0, 0)),
            pl.BlockSpec((BB, NUM_CLASSES), lambda i: (i, 0)),
            pl.BlockSpec((E, C), lambda i: (0, 0)),
            pl.BlockSpec((E, NUM_CLASSES), lambda i: (0, 0)),
            pl.BlockSpec((1, E), lambda i: (0, 0)),
        ],
        out_specs=[
            pl.BlockSpec((BB, E), lambda i: (i, 0)),
            pl.BlockSpec((BB, TOP_K), lambda i: (i, 0)),
            pl.BlockSpec((BB, TOP_K), lambda i: (i, 0)),
            pl.BlockSpec((1, E, TOP_K, BB), lambda i: (i, 0, 0, 0)),
        ],
        out_shape=[
            jax.ShapeDtypeStruct((B, E), jnp.float32),
            jax.ShapeDtypeStruct((B, TOP_K), jnp.float32),
            jax.ShapeDtypeStruct((B, TOP_K), jnp.int32),
            jax.ShapeDtypeStruct((B // BB, E, TOP_K, BB), jnp.int32),
        ],
        interpret=interpret,
    )(hs3, task_cls, w1, w2, b2)


def kernel(hidden_states, task_cls, W, b):
    hs3 = hidden_states.reshape(B, C, HW)
    w1 = W[:, :C]
    w2 = W[:, C:]
    b2 = b.reshape(1, E)
    logits, weights, sel, mask4 = _run(hs3, task_cls, w1, w2, b2)
    mask = mask4.transpose(1, 2, 0, 3).reshape(E, TOP_K, B)
    return (logits, weights, sel, mask)
